# baseline (speedup 1.0000x reference)
import functools

import jax
import jax.numpy as jnp
from jax import lax
from jax.experimental import pallas as pl
from jax.experimental.pallas import tpu as pltpu
from jax.experimental.pallas import tpu_sc as plsc

BATCH = 16384
D = 64
NW = 32
PER_W = BATCH // NW
CHUNK = 64
NCHUNK = PER_W // CHUNK
GROUPS = CHUNK // 16

SROWS = 50176
HOFF = 49920
TBLK = 256


def _tr_body(uL, uH, iL, iH, rL, rH, sL, sH, uo, io, ro, so):
    for lo, hi, o in ((uL, uH, uo), (iL, iH, io), (rL, rH, ro), (sL, sH, so)):
        o[:, 0:D] = lo[...].T
        o[:, D:2 * D] = hi[...].T


def _transpose_tables(uT, iT, rT, sT):
    lo_spec = pl.BlockSpec((D, TBLK), lambda t: (0, t))
    hi_spec = pl.BlockSpec((D, TBLK), lambda t: (0, t + HOFF // TBLK))
    out_spec = pl.BlockSpec((TBLK, 2 * D), lambda t: (t, 0))
    out_t = jax.ShapeDtypeStruct((SROWS, 2 * D), jnp.float32)
    return pl.pallas_call(
        _tr_body,
        grid=(SROWS // TBLK,),
        in_specs=[lo_spec, hi_spec] * 4,
        out_specs=[out_spec] * 4,
        out_shape=[out_t] * 4,
        compiler_params=pltpu.CompilerParams(
            dimension_semantics=("arbitrary",)),
    )(uT, uT, iT, iT, rT, rT, sT, sT)


def _mud_body(users, items, uE2, iE2, uBias, itemBias, gBias16, price,
              rU2, rI2, out,
              u_idx, i_idx, su_idx, si_idx, uB_v, iB_v, p_v, g_v, out_v,
              uE_b0, iE_b0, rU_b0, rI_b0, uE_b1, iE_b1, rU_b1, rI_b1,
              sem0, sem1, sem_small):
    c = lax.axis_index("c")
    s = lax.axis_index("s")
    wid = s * 2 + c
    base = wid * PER_W

    bufs = ((uE_b0, iE_b0, rU_b0, rI_b0), (uE_b1, iE_b1, rU_b1, rI_b1))
    sems = (sem0, sem1)

    pltpu.sync_copy(users.at[pl.ds(base, PER_W)], u_idx)
    pltpu.sync_copy(items.at[pl.ds(base, PER_W)], i_idx)

    def shift_body(g, carry):
        goff = pl.multiple_of(g * 16, 16)
        u16 = u_idx[pl.ds(goff, 16)]
        i16 = i_idx[pl.ds(goff, 16)]
        su_idx[pl.ds(goff, 16)] = u16 - jnp.where(u16 >= SROWS, HOFF, 0)
        si_idx[pl.ds(goff, 16)] = i16 - jnp.where(i16 >= SROWS, HOFF, 0)
        return carry

    lax.fori_loop(0, PER_W // 16, shift_body, 0)

    d_g = pltpu.async_copy(gBias16, g_v, sem_small)
    d_ub = pltpu.async_copy(uBias.at[u_idx], uB_v, sem_small)
    d_ib = pltpu.async_copy(itemBias.at[i_idx], iB_v, sem_small)
    d_p = pltpu.async_copy(price.at[i_idx], p_v, sem_small)

    def fire(ch):
        slot = ch % 2
        suidx_c = su_idx.at[pl.ds(ch * CHUNK, CHUNK)]
        siidx_c = si_idx.at[pl.ds(ch * CHUNK, CHUNK)]
        b = bufs[slot]
        sem = sems[slot]
        return (pltpu.async_copy(uE2.at[suidx_c], b[0], sem),
                pltpu.async_copy(iE2.at[siidx_c], b[1], sem),
                pltpu.async_copy(rU2.at[suidx_c], b[2], sem),
                pltpu.async_copy(rI2.at[siidx_c], b[3], sem))

    pend = fire(0)
    d_g.wait()
    d_ub.wait()
    d_ib.wait()
    d_p.wait()

    lane = lax.iota(jnp.int32, 16)

    for ch in range(NCHUNK):
        nxt = fire(ch + 1) if ch + 1 < NCHUNK else None
        for dsc in pend:
            dsc.wait()
        uE_b, iE_b, rU_b, rI_b = bufs[ch % 2]

        def group_body(g, carry, uE_b=uE_b, iE_b=iE_b, rU_b=rU_b,
                       rI_b=rI_b, ch=ch):
            rows = lane + g * 16
            goff = pl.multiple_of(ch * CHUNK + g * 16, 16)
            u16 = u_idx[pl.ds(goff, 16)]
            i16 = i_idx[pl.ds(goff, 16)]
            ucol0 = jnp.where(u16 >= SROWS, D, 0)
            icol0 = jnp.where(i16 >= SROWS, D, 0)
            acc_a = jnp.zeros((16,), jnp.float32)
            acc_r = jnp.zeros((16,), jnp.float32)
            for j in range(D):
                t = (lane + j) & (D - 1)
                ucol = ucol0 | t
                icol = icol0 | t
                ue = plsc.load_gather(uE_b, [rows, ucol])
                ie = plsc.load_gather(iE_b, [rows, icol])
                ru = plsc.load_gather(rU_b, [rows, ucol])
                ri = plsc.load_gather(rI_b, [rows, icol])
                acc_a = acc_a + ue * ie
                acc_r = acc_r + ru * ri
            ub = uB_v[pl.ds(goff, 16)]
            ib = iB_v[pl.ds(goff, 16)]
            pv = p_v[pl.ds(goff, 16)]
            alpha = g_v[...] + ub + ib + acc_a
            e = jnp.exp(-2.0 * jnp.abs(acc_r))
            th = jnp.sign(acc_r) * (1.0 - e) / (1.0 + e)
            res = (0.5 * alpha * th) * (1.0 + jnp.exp(-pv))
            out_v[pl.ds(goff, 16)] = res
            return carry

        lax.fori_loop(0, GROUPS, group_body, 0)
        pend = nxt

    pltpu.sync_copy(out_v, out.at[pl.ds(base, PER_W)])


def kernel(users, items, uEmbed, itemEmbed, uBias, itemBias, gBias, price, rU, rI):
    mesh = plsc.VectorSubcoreMesh(core_axis_name="c", subcore_axis_name="s")
    run = pl.kernel(
        _mud_body,
        out_type=jax.ShapeDtypeStruct((BATCH,), jnp.float32),
        mesh=mesh,
        compiler_params=pltpu.CompilerParams(
            use_tc_tiling_on_sc=True, needs_layout_passes=False
        ),
        scratch_types=[
            pltpu.VMEM((PER_W,), jnp.int32),
            pltpu.VMEM((PER_W,), jnp.int32),
            pltpu.VMEM((PER_W,), jnp.int32),
            pltpu.VMEM((PER_W,), jnp.int32),
            pltpu.VMEM((PER_W,), jnp.float32),
            pltpu.VMEM((PER_W,), jnp.float32),
            pltpu.VMEM((PER_W,), jnp.float32),
            pltpu.VMEM((16,), jnp.float32),
            pltpu.VMEM((PER_W,), jnp.float32),
            pltpu.VMEM((CHUNK, 2 * D), jnp.float32),
            pltpu.VMEM((CHUNK, 2 * D), jnp.float32),
            pltpu.VMEM((CHUNK, 2 * D), jnp.float32),
            pltpu.VMEM((CHUNK, 2 * D), jnp.float32),
            pltpu.VMEM((CHUNK, 2 * D), jnp.float32),
            pltpu.VMEM((CHUNK, 2 * D), jnp.float32),
            pltpu.VMEM((CHUNK, 2 * D), jnp.float32),
            pltpu.VMEM((CHUNK, 2 * D), jnp.float32),
            pltpu.SemaphoreType.DMA,
            pltpu.SemaphoreType.DMA,
            pltpu.SemaphoreType.DMA,
        ],
    )
    g16 = jnp.broadcast_to(gBias.reshape(1), (16,))
    uE2, iE2, rU2, rI2 = _transpose_tables(
        uEmbed.T, itemEmbed.T, rU.T, rI.T)
    return run(users.astype(jnp.int32), items.astype(jnp.int32),
               uE2, iE2, uBias.reshape(-1), itemBias.reshape(-1),
               g16, price, rU2, rI2)


# speedup vs baseline: 1.2548x; 1.2548x over previous
import functools

import jax
import jax.numpy as jnp
from jax import lax
from jax.experimental import pallas as pl
from jax.experimental.pallas import tpu as pltpu
from jax.experimental.pallas import tpu_sc as plsc

BATCH = 16384
D = 64
NW = 32
PER_W = BATCH // NW
CHUNK = 64
NCHUNK = PER_W // CHUNK
GROUPS = CHUNK // 16

SROWS = 50688
HOFF = 49664
TBLK = 512


def _tr_body(uL, uH, iL, iH, rL, rH, sL, sH, uo, io, ro, so):
    eye = jnp.eye(D, dtype=jnp.float32)
    dn = (((0,), (0,)), ((), ()))
    for lo, hi, o in ((uL, uH, uo), (iL, iH, io), (rL, rH, ro), (sL, sH, so)):
        o[:, 0:D] = lax.dot_general(lo[...], eye, dn,
                                    preferred_element_type=jnp.float32)
        o[:, D:2 * D] = lax.dot_general(hi[...], eye, dn,
                                        preferred_element_type=jnp.float32)


def _transpose_tables(uT, iT, rT, sT):
    lo_spec = pl.BlockSpec((D, TBLK), lambda t: (0, t))
    hi_spec = pl.BlockSpec((D, TBLK), lambda t: (0, t + HOFF // TBLK))
    out_spec = pl.BlockSpec((TBLK, 2 * D), lambda t: (t, 0))
    out_t = jax.ShapeDtypeStruct((SROWS, 2 * D), jnp.float32)
    return pl.pallas_call(
        _tr_body,
        grid=(SROWS // TBLK,),
        in_specs=[lo_spec, hi_spec] * 4,
        out_specs=[out_spec] * 4,
        out_shape=[out_t] * 4,
        compiler_params=pltpu.CompilerParams(
            dimension_semantics=("arbitrary",)),
    )(uT, uT, iT, iT, rT, rT, sT, sT)


def _mud_body(users, items, uE2, iE2, uBias, itemBias, gBias16, price,
              rU2, rI2, out,
              u_idx, i_idx, su_idx, si_idx, uB_v, iB_v, p_v, g_v, out_v,
              uE_b0, iE_b0, rU_b0, rI_b0, uE_b1, iE_b1, rU_b1, rI_b1,
              sem0, sem1, sem_small):
    c = lax.axis_index("c")
    s = lax.axis_index("s")
    wid = s * 2 + c
    base = wid * PER_W

    bufs = ((uE_b0, iE_b0, rU_b0, rI_b0), (uE_b1, iE_b1, rU_b1, rI_b1))
    sems = (sem0, sem1)

    pltpu.sync_copy(users.at[pl.ds(base, PER_W)], u_idx)
    pltpu.sync_copy(items.at[pl.ds(base, PER_W)], i_idx)

    def shift_body(g, carry):
        goff = pl.multiple_of(g * 16, 16)
        u16 = u_idx[pl.ds(goff, 16)]
        i16 = i_idx[pl.ds(goff, 16)]
        su_idx[pl.ds(goff, 16)] = u16 - jnp.where(u16 >= SROWS, HOFF, 0)
        si_idx[pl.ds(goff, 16)] = i16 - jnp.where(i16 >= SROWS, HOFF, 0)
        return carry

    lax.fori_loop(0, PER_W // 16, shift_body, 0)

    d_g = pltpu.async_copy(gBias16, g_v, sem_small)
    d_ub = pltpu.async_copy(uBias.at[u_idx], uB_v, sem_small)
    d_ib = pltpu.async_copy(itemBias.at[i_idx], iB_v, sem_small)
    d_p = pltpu.async_copy(price.at[i_idx], p_v, sem_small)

    def fire(ch):
        slot = ch % 2
        suidx_c = su_idx.at[pl.ds(ch * CHUNK, CHUNK)]
        siidx_c = si_idx.at[pl.ds(ch * CHUNK, CHUNK)]
        b = bufs[slot]
        sem = sems[slot]
        return (pltpu.async_copy(uE2.at[suidx_c], b[0], sem),
                pltpu.async_copy(iE2.at[siidx_c], b[1], sem),
                pltpu.async_copy(rU2.at[suidx_c], b[2], sem),
                pltpu.async_copy(rI2.at[siidx_c], b[3], sem))

    pend = fire(0)
    d_g.wait()
    d_ub.wait()
    d_ib.wait()
    d_p.wait()

    lane = lax.iota(jnp.int32, 16)

    for ch in range(NCHUNK):
        nxt = fire(ch + 1) if ch + 1 < NCHUNK else None
        for dsc in pend:
            dsc.wait()
        uE_b, iE_b, rU_b, rI_b = bufs[ch % 2]

        def group_body(g, carry, uE_b=uE_b, iE_b=iE_b, rU_b=rU_b,
                       rI_b=rI_b, ch=ch):
            rows = lane + g * 16
            goff = pl.multiple_of(ch * CHUNK + g * 16, 16)
            u16 = u_idx[pl.ds(goff, 16)]
            i16 = i_idx[pl.ds(goff, 16)]
            ucol0 = jnp.where(u16 >= SROWS, D, 0)
            icol0 = jnp.where(i16 >= SROWS, D, 0)
            acc_a = jnp.zeros((16,), jnp.float32)
            acc_r = jnp.zeros((16,), jnp.float32)
            for j in range(D):
                t = (lane + j) & (D - 1)
                ucol = ucol0 | t
                icol = icol0 | t
                ue = plsc.load_gather(uE_b, [rows, ucol])
                ie = plsc.load_gather(iE_b, [rows, icol])
                ru = plsc.load_gather(rU_b, [rows, ucol])
                ri = plsc.load_gather(rI_b, [rows, icol])
                acc_a = acc_a + ue * ie
                acc_r = acc_r + ru * ri
            ub = uB_v[pl.ds(goff, 16)]
            ib = iB_v[pl.ds(goff, 16)]
            pv = p_v[pl.ds(goff, 16)]
            alpha = g_v[...] + ub + ib + acc_a
            e = jnp.exp(-2.0 * jnp.abs(acc_r))
            th = jnp.sign(acc_r) * (1.0 - e) / (1.0 + e)
            res = (0.5 * alpha * th) * (1.0 + jnp.exp(-pv))
            out_v[pl.ds(goff, 16)] = res
            return carry

        lax.fori_loop(0, GROUPS, group_body, 0)
        pend = nxt

    pltpu.sync_copy(out_v, out.at[pl.ds(base, PER_W)])


def kernel(users, items, uEmbed, itemEmbed, uBias, itemBias, gBias, price, rU, rI):
    mesh = plsc.VectorSubcoreMesh(core_axis_name="c", subcore_axis_name="s")
    run = pl.kernel(
        _mud_body,
        out_type=jax.ShapeDtypeStruct((BATCH,), jnp.float32),
        mesh=mesh,
        compiler_params=pltpu.CompilerParams(
            use_tc_tiling_on_sc=True, needs_layout_passes=False
        ),
        scratch_types=[
            pltpu.VMEM((PER_W,), jnp.int32),
            pltpu.VMEM((PER_W,), jnp.int32),
            pltpu.VMEM((PER_W,), jnp.int32),
            pltpu.VMEM((PER_W,), jnp.int32),
            pltpu.VMEM((PER_W,), jnp.float32),
            pltpu.VMEM((PER_W,), jnp.float32),
            pltpu.VMEM((PER_W,), jnp.float32),
            pltpu.VMEM((16,), jnp.float32),
            pltpu.VMEM((PER_W,), jnp.float32),
            pltpu.VMEM((CHUNK, 2 * D), jnp.float32),
            pltpu.VMEM((CHUNK, 2 * D), jnp.float32),
            pltpu.VMEM((CHUNK, 2 * D), jnp.float32),
            pltpu.VMEM((CHUNK, 2 * D), jnp.float32),
            pltpu.VMEM((CHUNK, 2 * D), jnp.float32),
            pltpu.VMEM((CHUNK, 2 * D), jnp.float32),
            pltpu.VMEM((CHUNK, 2 * D), jnp.float32),
            pltpu.VMEM((CHUNK, 2 * D), jnp.float32),
            pltpu.SemaphoreType.DMA,
            pltpu.SemaphoreType.DMA,
            pltpu.SemaphoreType.DMA,
        ],
    )
    g16 = jnp.broadcast_to(gBias.reshape(1), (16,))
    uE2, iE2, rU2, rI2 = _transpose_tables(
        uEmbed.T, itemEmbed.T, rU.T, rI.T)
    return run(users.astype(jnp.int32), items.astype(jnp.int32),
               uE2, iE2, uBias.reshape(-1), itemBias.reshape(-1),
               g16, price, rU2, rI2)


# speedup vs baseline: 1.6508x; 1.3156x over previous
import functools

import jax
import jax.numpy as jnp
from jax import lax
from jax.experimental import pallas as pl
from jax.experimental.pallas import tpu as pltpu
from jax.experimental.pallas import tpu_sc as plsc

BATCH = 16384
D = 64
NW = 32
PER_W = BATCH // NW
CHUNK = 64
NCHUNK = PER_W // CHUNK
GROUPS = CHUNK // 16

SROWS = 51200
HOFF = 49152
TBLK = 2048


def _tr_body(uL, uH, iL, iH, rL, rH, sL, sH, uo, io, ro, so):
    eye = jnp.eye(D, dtype=jnp.float32)
    dn = (((0,), (0,)), ((), ()))
    for lo, hi, o in ((uL, uH, uo), (iL, iH, io), (rL, rH, ro), (sL, sH, so)):
        o[:, 0:D] = lax.dot_general(lo[...], eye, dn,
                                    preferred_element_type=jnp.float32)
        o[:, D:2 * D] = lax.dot_general(hi[...], eye, dn,
                                        preferred_element_type=jnp.float32)


def _transpose_tables(uT, iT, rT, sT):
    lo_spec = pl.BlockSpec((D, TBLK), lambda t: (0, t))
    hi_spec = pl.BlockSpec((D, TBLK), lambda t: (0, t + HOFF // TBLK))
    out_spec = pl.BlockSpec((TBLK, 2 * D), lambda t: (t, 0))
    out_t = jax.ShapeDtypeStruct((SROWS, 2 * D), jnp.float32)
    return pl.pallas_call(
        _tr_body,
        grid=(SROWS // TBLK,),
        in_specs=[lo_spec, hi_spec] * 4,
        out_specs=[out_spec] * 4,
        out_shape=[out_t] * 4,
        compiler_params=pltpu.CompilerParams(
            dimension_semantics=("arbitrary",)),
    )(uT, uT, iT, iT, rT, rT, sT, sT)


def _mud_body(users, items, uE2, iE2, uBias, itemBias, gBias16, price,
              rU2, rI2, out,
              u_idx, i_idx, su_idx, si_idx, uB_v, iB_v, p_v, g_v, out_v,
              uE_b0, iE_b0, rU_b0, rI_b0, uE_b1, iE_b1, rU_b1, rI_b1,
              sem0, sem1, sem_small):
    c = lax.axis_index("c")
    s = lax.axis_index("s")
    wid = s * 2 + c
    base = wid * PER_W

    bufs = ((uE_b0, iE_b0, rU_b0, rI_b0), (uE_b1, iE_b1, rU_b1, rI_b1))
    sems = (sem0, sem1)

    pltpu.sync_copy(users.at[pl.ds(base, PER_W)], u_idx)
    pltpu.sync_copy(items.at[pl.ds(base, PER_W)], i_idx)

    def shift_body(g, carry):
        goff = pl.multiple_of(g * 16, 16)
        u16 = u_idx[pl.ds(goff, 16)]
        i16 = i_idx[pl.ds(goff, 16)]
        su_idx[pl.ds(goff, 16)] = u16 - jnp.where(u16 >= SROWS, HOFF, 0)
        si_idx[pl.ds(goff, 16)] = i16 - jnp.where(i16 >= SROWS, HOFF, 0)
        return carry

    lax.fori_loop(0, PER_W // 16, shift_body, 0)

    d_g = pltpu.async_copy(gBias16, g_v, sem_small)
    d_ub = pltpu.async_copy(uBias.at[u_idx], uB_v, sem_small)
    d_ib = pltpu.async_copy(itemBias.at[i_idx], iB_v, sem_small)
    d_p = pltpu.async_copy(price.at[i_idx], p_v, sem_small)

    def fire(ch):
        slot = ch % 2
        suidx_c = su_idx.at[pl.ds(ch * CHUNK, CHUNK)]
        siidx_c = si_idx.at[pl.ds(ch * CHUNK, CHUNK)]
        b = bufs[slot]
        sem = sems[slot]
        return (pltpu.async_copy(uE2.at[suidx_c], b[0], sem),
                pltpu.async_copy(iE2.at[siidx_c], b[1], sem),
                pltpu.async_copy(rU2.at[suidx_c], b[2], sem),
                pltpu.async_copy(rI2.at[siidx_c], b[3], sem))

    pend = fire(0)
    d_g.wait()
    d_ub.wait()
    d_ib.wait()
    d_p.wait()

    lane = lax.iota(jnp.int32, 16)

    for ch in range(NCHUNK):
        nxt = fire(ch + 1) if ch + 1 < NCHUNK else None
        for dsc in pend:
            dsc.wait()
        uE_b, iE_b, rU_b, rI_b = bufs[ch % 2]

        def group_body(g, carry, uE_b=uE_b, iE_b=iE_b, rU_b=rU_b,
                       rI_b=rI_b, ch=ch):
            rows = lane + g * 16
            goff = pl.multiple_of(ch * CHUNK + g * 16, 16)
            u16 = u_idx[pl.ds(goff, 16)]
            i16 = i_idx[pl.ds(goff, 16)]
            ucol0 = jnp.where(u16 >= SROWS, D, 0)
            icol0 = jnp.where(i16 >= SROWS, D, 0)
            acc_a = jnp.zeros((16,), jnp.float32)
            acc_r = jnp.zeros((16,), jnp.float32)
            for j in range(D):
                t = (lane + j) & (D - 1)
                ucol = ucol0 | t
                icol = icol0 | t
                ue = plsc.load_gather(uE_b, [rows, ucol])
                ie = plsc.load_gather(iE_b, [rows, icol])
                ru = plsc.load_gather(rU_b, [rows, ucol])
                ri = plsc.load_gather(rI_b, [rows, icol])
                acc_a = acc_a + ue * ie
                acc_r = acc_r + ru * ri
            ub = uB_v[pl.ds(goff, 16)]
            ib = iB_v[pl.ds(goff, 16)]
            pv = p_v[pl.ds(goff, 16)]
            alpha = g_v[...] + ub + ib + acc_a
            e = jnp.exp(-2.0 * jnp.abs(acc_r))
            th = jnp.sign(acc_r) * (1.0 - e) / (1.0 + e)
            res = (0.5 * alpha * th) * (1.0 + jnp.exp(-pv))
            out_v[pl.ds(goff, 16)] = res
            return carry

        lax.fori_loop(0, GROUPS, group_body, 0)
        pend = nxt

    pltpu.sync_copy(out_v, out.at[pl.ds(base, PER_W)])


def kernel(users, items, uEmbed, itemEmbed, uBias, itemBias, gBias, price, rU, rI):
    mesh = plsc.VectorSubcoreMesh(core_axis_name="c", subcore_axis_name="s")
    run = pl.kernel(
        _mud_body,
        out_type=jax.ShapeDtypeStruct((BATCH,), jnp.float32),
        mesh=mesh,
        compiler_params=pltpu.CompilerParams(
            use_tc_tiling_on_sc=True, needs_layout_passes=False
        ),
        scratch_types=[
            pltpu.VMEM((PER_W,), jnp.int32),
            pltpu.VMEM((PER_W,), jnp.int32),
            pltpu.VMEM((PER_W,), jnp.int32),
            pltpu.VMEM((PER_W,), jnp.int32),
            pltpu.VMEM((PER_W,), jnp.float32),
            pltpu.VMEM((PER_W,), jnp.float32),
            pltpu.VMEM((PER_W,), jnp.float32),
            pltpu.VMEM((16,), jnp.float32),
            pltpu.VMEM((PER_W,), jnp.float32),
            pltpu.VMEM((CHUNK, 2 * D), jnp.float32),
            pltpu.VMEM((CHUNK, 2 * D), jnp.float32),
            pltpu.VMEM((CHUNK, 2 * D), jnp.float32),
            pltpu.VMEM((CHUNK, 2 * D), jnp.float32),
            pltpu.VMEM((CHUNK, 2 * D), jnp.float32),
            pltpu.VMEM((CHUNK, 2 * D), jnp.float32),
            pltpu.VMEM((CHUNK, 2 * D), jnp.float32),
            pltpu.VMEM((CHUNK, 2 * D), jnp.float32),
            pltpu.SemaphoreType.DMA,
            pltpu.SemaphoreType.DMA,
            pltpu.SemaphoreType.DMA,
        ],
    )
    g16 = jnp.broadcast_to(gBias.reshape(1), (16,))
    uE2, iE2, rU2, rI2 = _transpose_tables(
        uEmbed.T, itemEmbed.T, rU.T, rI.T)
    return run(users.astype(jnp.int32), items.astype(jnp.int32),
               uE2, iE2, uBias.reshape(-1), itemBias.reshape(-1),
               g16, price, rU2, rI2)
